# baseline (speedup 1.0000x reference)
import functools

import jax
import jax.numpy as jnp
from jax import lax
from jax.experimental import pallas as pl
from jax.experimental.pallas import tpu as pltpu
from jax.experimental.pallas import tpu_sc as plsc

_CHUNK = 32
_NBUF = 4


def kernel(x, emb_weight):
    sl = x.shape[1]
    dim = emb_weight.shape[1]
    info = plsc.get_sparse_core_info()
    nc, ns = info.num_cores, info.num_subcores
    nw = nc * ns
    rows_per_w = sl // nw
    nchunks = rows_per_w // _CHUNK

    mesh = plsc.VectorSubcoreMesh(core_axis_name="c", subcore_axis_name="s")

    @functools.partial(
        pl.kernel,
        mesh=mesh,
        out_type=jax.ShapeDtypeStruct((sl, dim), emb_weight.dtype),
        scratch_types=(
            [pltpu.VMEM((_CHUNK, dim), jnp.float32) for _ in range(_NBUF)]
            + [pltpu.SemaphoreType.DMA for _ in range(2 * _NBUF)]
        ),
    )
    def copy_k(emb_hbm, out_hbm, *scratch):
        bufs = scratch[:_NBUF]
        isems = scratch[_NBUF : 2 * _NBUF]
        osems = scratch[2 * _NBUF :]
        wid = lax.axis_index("s") * nc + lax.axis_index("c")
        base = wid * rows_per_w

        def load(i):
            b = i % _NBUF
            return pltpu.async_copy(
                emb_hbm.at[pl.ds(base + i * _CHUNK, _CHUNK)], bufs[b], isems[b]
            )

        def store(i):
            b = i % _NBUF
            return pltpu.async_copy(
                bufs[b], out_hbm.at[pl.ds(base + i * _CHUNK, _CHUNK)], osems[b]
            )

        loads = {}
        stores = {}
        for i in range(min(_NBUF, nchunks)):
            loads[i] = load(i)
        for i in range(nchunks):
            if i >= _NBUF:
                stores[i - _NBUF].wait()
                loads[i] = load(i)
            loads[i].wait()
            stores[i] = store(i)
        for i in range(max(0, nchunks - _NBUF), nchunks):
            stores[i].wait()

    return copy_k(emb_weight)
